# baseline (speedup 1.0000x reference)
import functools

import jax
import jax.numpy as jnp
from jax import lax
from jax.experimental import pallas as pl
from jax.experimental.pallas import tpu as pltpu
from jax.experimental.pallas import tpu_sc as plsc

N = 10000
D = 128
E = 320000
NC = 2
NS = 16
NW = NC * NS
EPW = E // NW
CH = 80
NCHUNK = EPW // CH
RPS = 624
RTAIL = N - NS * RPS

_BLK = 1000


def _dual_mm_body(x_ref, wa_ref, wb_ref, a_ref, b_ref):
    x = x_ref[...]
    a_ref[...] = lax.dot_general(
        x, wa_ref[...], (((1,), (1,)), ((), ())),
        preferred_element_type=jnp.float32)
    b_ref[...] = lax.dot_general(
        x, wb_ref[...], (((1,), (1,)), ((), ())),
        preferred_element_type=jnp.float32)


def _dual_matmul_t(x, wa, wb):
    return pl.pallas_call(
        _dual_mm_body,
        grid=(N // _BLK,),
        in_specs=[pl.BlockSpec((_BLK, D), lambda i: (i, 0)),
                  pl.BlockSpec((D, D), lambda i: (0, 0)),
                  pl.BlockSpec((D, D), lambda i: (0, 0))],
        out_specs=[pl.BlockSpec((_BLK, D), lambda i: (i, 0)),
                   pl.BlockSpec((_BLK, D), lambda i: (i, 0))],
        out_shape=[jax.ShapeDtypeStruct((N, D), jnp.float32),
                   jax.ShapeDtypeStruct((N, D), jnp.float32)],
    )(x, wa, wb)


def _mid_body(agg_ref, r_ref, b_ref, wrel2_ref, wroot2_ref, y2_ref, r2_ref):
    h = jnp.maximum(agg_ref[0] + agg_ref[1] + b_ref[...] + r_ref[...], 0.0)
    y2_ref[...] = lax.dot_general(
        h, wrel2_ref[...], (((1,), (1,)), ((), ())),
        preferred_element_type=jnp.float32)
    r2_ref[...] = lax.dot_general(
        h, wroot2_ref[...], (((1,), (1,)), ((), ())),
        preferred_element_type=jnp.float32)


def _mid_stage(aggp, r, b, wrel2, wroot2):
    return pl.pallas_call(
        _mid_body,
        grid=(N // _BLK,),
        in_specs=[pl.BlockSpec((2, _BLK, D), lambda i: (0, i, 0)),
                  pl.BlockSpec((_BLK, D), lambda i: (i, 0)),
                  pl.BlockSpec((1, D), lambda i: (0, 0)),
                  pl.BlockSpec((D, D), lambda i: (0, 0)),
                  pl.BlockSpec((D, D), lambda i: (0, 0))],
        out_specs=[pl.BlockSpec((_BLK, D), lambda i: (i, 0)),
                   pl.BlockSpec((_BLK, D), lambda i: (i, 0))],
        out_shape=[jax.ShapeDtypeStruct((N, D), jnp.float32),
                   jax.ShapeDtypeStruct((N, D), jnp.float32)],
    )(aggp, r, b, wrel2, wroot2)


def _final_body(agg_ref, r_ref, b_ref, o_ref):
    o_ref[...] = agg_ref[0] + agg_ref[1] + b_ref[...] + r_ref[...]


def _final_stage(aggp, r, b):
    return pl.pallas_call(
        _final_body,
        grid=(N // _BLK,),
        in_specs=[pl.BlockSpec((2, _BLK, D), lambda i: (0, i, 0)),
                  pl.BlockSpec((_BLK, D), lambda i: (i, 0)),
                  pl.BlockSpec((1, D), lambda i: (0, 0))],
        out_specs=pl.BlockSpec((_BLK, D), lambda i: (i, 0)),
        out_shape=jax.ShapeDtypeStruct((N, D), jnp.float32),
    )(aggp, r, b)


def _sc_segment_sum(y, src, dst, zeros):
    mesh = plsc.VectorSubcoreMesh(core_axis_name="c", subcore_axis_name="s")

    @functools.partial(
        pl.kernel, mesh=mesh,
        out_type=jax.ShapeDtypeStruct((NC, N, D), jnp.float32),
        scratch_types=[
            pltpu.VMEM((EPW,), jnp.int32),
            pltpu.VMEM((NCHUNK, CH), jnp.int32),
            pltpu.VMEM((CH, D), jnp.float32),
            pltpu.VMEM((CH, D), jnp.float32),
            pltpu.VMEM_SHARED((N, D), jnp.float32),
            pltpu.SemaphoreType.DMA,
            pltpu.SemaphoreType.DMA,
        ],
    )
    def scat(y_hbm, src_hbm, dst_hbm, zero_hbm, out_hbm,
             src_v, dst_v, rows0, rows1, acc, sem0, sem1):
        c = lax.axis_index("c")
        s = lax.axis_index("s")
        wid = s * NC + c
        cp_src = pltpu.async_copy(src_hbm.at[wid], src_v, sem1)
        cp_dst = pltpu.async_copy(dst_hbm.at[wid], dst_v, sem1)
        cp_z = pltpu.async_copy(zero_hbm.at[pl.ds(s * RPS, RPS)],
                                acc.at[pl.ds(s * RPS, RPS)], sem1)

        @pl.when(s == 0)
        def _():
            pltpu.async_copy(zero_hbm.at[pl.ds(NS * RPS, RTAIL)],
                             acc.at[pl.ds(NS * RPS, RTAIL)], sem1).wait()

        cp_src.wait()
        cp_dst.wait()
        cp_z.wait()
        plsc.subcore_barrier()

        pltpu.async_copy(y_hbm.at[src_v.at[pl.ds(0, CH)]], rows0, sem0)
        pltpu.async_copy(y_hbm.at[src_v.at[pl.ds(CH, CH)]], rows1, sem1)

        @pl.loop(0, NCHUNK - 1, step=2)
        def _(j):
            pltpu.make_async_copy(
                y_hbm.at[src_v.at[pl.ds(j * CH, CH)]], rows0, sem0).wait()
            pltpu.sync_copy(rows0, acc.at[dst_v.at[j]], add=True)
            pltpu.async_copy(
                y_hbm.at[src_v.at[pl.ds((j + 2) * CH, CH)]], rows0, sem0)
            pltpu.make_async_copy(
                y_hbm.at[src_v.at[pl.ds((j + 1) * CH, CH)]], rows1, sem1).wait()
            pltpu.sync_copy(rows1, acc.at[dst_v.at[j + 1]], add=True)

            @pl.when(j + 3 < NCHUNK)
            def _():
                pltpu.async_copy(
                    y_hbm.at[src_v.at[pl.ds((j + 3) * CH, CH)]], rows1, sem1)

        pltpu.make_async_copy(
            y_hbm.at[src_v.at[pl.ds((NCHUNK - 1) * CH, CH)]],
            rows0, sem0).wait()
        pltpu.sync_copy(rows0, acc.at[dst_v.at[NCHUNK - 1]], add=True)

        plsc.subcore_barrier()
        pltpu.sync_copy(acc.at[pl.ds(s * RPS, RPS)],
                        out_hbm.at[c].at[pl.ds(s * RPS, RPS)])

        @pl.when(s == 0)
        def _():
            pltpu.sync_copy(acc.at[pl.ds(NS * RPS, RTAIL)],
                            out_hbm.at[c].at[pl.ds(NS * RPS, RTAIL)])

    return scat(y, src, dst, zeros)


def kernel(x, edge_index, W1_rel, b1_rel, W1_root, W2_rel, b2_rel, W2_root):
    ei = edge_index.astype(jnp.int32)
    src = ei[0].reshape(NW, EPW)
    dst = ei[1].reshape(NW, NCHUNK, CH)
    zeros = jnp.zeros((N, D), jnp.float32)
    b1 = b1_rel.reshape(1, D)
    b2 = b2_rel.reshape(1, D)

    y1, r1 = _dual_matmul_t(x, W1_rel, W1_root)
    agg1 = _sc_segment_sum(y1, src, dst, zeros)
    y2, r2 = _mid_stage(agg1, r1, b1, W2_rel, W2_root)
    agg2 = _sc_segment_sum(y2, src, dst, zeros)
    out = _final_stage(agg2, r2, b2)
    return out


# speedup vs baseline: 1.0340x; 1.0340x over previous
import functools

import jax
import jax.numpy as jnp
from jax import lax
from jax.experimental import pallas as pl
from jax.experimental.pallas import tpu as pltpu
from jax.experimental.pallas import tpu_sc as plsc

N = 10000
D = 128
E = 320000
NC = 2
NS = 16
NW = NC * NS
EPW = E // NW
CH = 80
NCHUNK = EPW // CH
RPS = 624
RTAIL = N - NS * RPS

_BLK = 2000


def _dual_mm_body(x_ref, wa_ref, wb_ref, a_ref, b_ref):
    x = x_ref[...]
    a_ref[...] = lax.dot_general(
        x, wa_ref[...], (((1,), (1,)), ((), ())),
        preferred_element_type=jnp.float32)
    b_ref[...] = lax.dot_general(
        x, wb_ref[...], (((1,), (1,)), ((), ())),
        preferred_element_type=jnp.float32)


def _dual_matmul_t(x, wa, wb):
    return pl.pallas_call(
        _dual_mm_body,
        grid=(N // _BLK,),
        in_specs=[pl.BlockSpec((_BLK, D), lambda i: (i, 0)),
                  pl.BlockSpec((D, D), lambda i: (0, 0)),
                  pl.BlockSpec((D, D), lambda i: (0, 0))],
        out_specs=[pl.BlockSpec((_BLK, D), lambda i: (i, 0)),
                   pl.BlockSpec((_BLK, D), lambda i: (i, 0))],
        out_shape=[jax.ShapeDtypeStruct((N, D), jnp.float32),
                   jax.ShapeDtypeStruct((N, D), jnp.float32)],
    )(x, wa, wb)


def _mid_body(agg_ref, r_ref, b_ref, wrel2_ref, wroot2_ref, y2_ref, r2_ref):
    h = jnp.maximum(agg_ref[0] + agg_ref[1] + b_ref[...] + r_ref[...], 0.0)
    y2_ref[...] = lax.dot_general(
        h, wrel2_ref[...], (((1,), (1,)), ((), ())),
        preferred_element_type=jnp.float32)
    r2_ref[...] = lax.dot_general(
        h, wroot2_ref[...], (((1,), (1,)), ((), ())),
        preferred_element_type=jnp.float32)


def _mid_stage(aggp, r, b, wrel2, wroot2):
    return pl.pallas_call(
        _mid_body,
        grid=(N // _BLK,),
        in_specs=[pl.BlockSpec((2, _BLK, D), lambda i: (0, i, 0)),
                  pl.BlockSpec((_BLK, D), lambda i: (i, 0)),
                  pl.BlockSpec((1, D), lambda i: (0, 0)),
                  pl.BlockSpec((D, D), lambda i: (0, 0)),
                  pl.BlockSpec((D, D), lambda i: (0, 0))],
        out_specs=[pl.BlockSpec((_BLK, D), lambda i: (i, 0)),
                   pl.BlockSpec((_BLK, D), lambda i: (i, 0))],
        out_shape=[jax.ShapeDtypeStruct((N, D), jnp.float32),
                   jax.ShapeDtypeStruct((N, D), jnp.float32)],
    )(aggp, r, b, wrel2, wroot2)


def _final_body(agg_ref, r_ref, b_ref, o_ref):
    o_ref[...] = agg_ref[0] + agg_ref[1] + b_ref[...] + r_ref[...]


def _final_stage(aggp, r, b):
    return pl.pallas_call(
        _final_body,
        grid=(N // _BLK,),
        in_specs=[pl.BlockSpec((2, _BLK, D), lambda i: (0, i, 0)),
                  pl.BlockSpec((_BLK, D), lambda i: (i, 0)),
                  pl.BlockSpec((1, D), lambda i: (0, 0))],
        out_specs=pl.BlockSpec((_BLK, D), lambda i: (i, 0)),
        out_shape=jax.ShapeDtypeStruct((N, D), jnp.float32),
    )(aggp, r, b)


def _sc_segment_sum(y, src, dst, zeros):
    mesh = plsc.VectorSubcoreMesh(core_axis_name="c", subcore_axis_name="s")

    @functools.partial(
        pl.kernel, mesh=mesh,
        out_type=jax.ShapeDtypeStruct((NC, N, D), jnp.float32),
        scratch_types=[
            pltpu.VMEM((EPW,), jnp.int32),
            pltpu.VMEM((NCHUNK, CH), jnp.int32),
            pltpu.VMEM((CH, D), jnp.float32),
            pltpu.VMEM((CH, D), jnp.float32),
            pltpu.VMEM_SHARED((N, D), jnp.float32),
            pltpu.SemaphoreType.DMA,
            pltpu.SemaphoreType.DMA,
            pltpu.SemaphoreType.DMA,
        ],
    )
    def scat(y_hbm, src_hbm, dst_hbm, zero_hbm, out_hbm,
             src_v, dst_v, rows0, rows1, acc, sem0, sem1, semp):
        c = lax.axis_index("c")
        s = lax.axis_index("s")
        wid = s * NC + c
        cp_src = pltpu.async_copy(src_hbm.at[wid], src_v, semp)
        cp_dst = pltpu.async_copy(dst_hbm.at[wid], dst_v, semp)
        cp_z = pltpu.async_copy(zero_hbm.at[pl.ds(s * RPS, RPS)],
                                acc.at[pl.ds(s * RPS, RPS)], semp)

        @pl.when(s == 0)
        def _():
            pltpu.async_copy(zero_hbm.at[pl.ds(NS * RPS, RTAIL)],
                             acc.at[pl.ds(NS * RPS, RTAIL)], semp).wait()

        cp_src.wait()
        pltpu.async_copy(y_hbm.at[src_v.at[pl.ds(0, CH)]], rows0, sem0)
        pltpu.async_copy(y_hbm.at[src_v.at[pl.ds(CH, CH)]], rows1, sem1)
        cp_dst.wait()
        cp_z.wait()
        plsc.subcore_barrier()


        @pl.loop(0, NCHUNK - 1, step=2)
        def _(j):
            pltpu.make_async_copy(
                y_hbm.at[src_v.at[pl.ds(j * CH, CH)]], rows0, sem0).wait()
            pltpu.sync_copy(rows0, acc.at[dst_v.at[j]], add=True)
            pltpu.async_copy(
                y_hbm.at[src_v.at[pl.ds((j + 2) * CH, CH)]], rows0, sem0)
            pltpu.make_async_copy(
                y_hbm.at[src_v.at[pl.ds((j + 1) * CH, CH)]], rows1, sem1).wait()
            pltpu.sync_copy(rows1, acc.at[dst_v.at[j + 1]], add=True)

            @pl.when(j + 3 < NCHUNK)
            def _():
                pltpu.async_copy(
                    y_hbm.at[src_v.at[pl.ds((j + 3) * CH, CH)]], rows1, sem1)

        pltpu.make_async_copy(
            y_hbm.at[src_v.at[pl.ds((NCHUNK - 1) * CH, CH)]],
            rows0, sem0).wait()
        pltpu.sync_copy(rows0, acc.at[dst_v.at[NCHUNK - 1]], add=True)

        plsc.subcore_barrier()
        pltpu.sync_copy(acc.at[pl.ds(s * RPS, RPS)],
                        out_hbm.at[c].at[pl.ds(s * RPS, RPS)])

        @pl.when(s == 0)
        def _():
            pltpu.sync_copy(acc.at[pl.ds(NS * RPS, RTAIL)],
                            out_hbm.at[c].at[pl.ds(NS * RPS, RTAIL)])

    return scat(y, src, dst, zeros)


def kernel(x, edge_index, W1_rel, b1_rel, W1_root, W2_rel, b2_rel, W2_root):
    ei = edge_index.astype(jnp.int32)
    src = ei[0].reshape(NW, EPW)
    dst = ei[1].reshape(NW, NCHUNK, CH)
    zeros = jnp.zeros((N, D), jnp.float32)
    b1 = b1_rel.reshape(1, D)
    b2 = b2_rel.reshape(1, D)

    y1, r1 = _dual_matmul_t(x, W1_rel, W1_root)
    agg1 = _sc_segment_sum(y1, src, dst, zeros)
    y2, r2 = _mid_stage(agg1, r1, b1, W2_rel, W2_root)
    agg2 = _sc_segment_sum(y2, src, dst, zeros)
    out = _final_stage(agg2, r2, b2)
    return out
